# baseline (speedup 1.0000x reference)
import functools

import jax
import jax.numpy as jnp
import numpy as np
from jax import lax
from jax.experimental import pallas as pl
from jax.experimental.pallas import tpu as pltpu
from jax.experimental.pallas import tpu_sc as plsc

NC, NS, L = 2, 16, 16
NW = NC * NS
B, S, D = 1024, 200, 64
CH = 200
HALVES = S // CH
NSEQ = B // NW
NBUF = 8
VECS = D // L
SCALE = float(np.sqrt(np.float32(D)))

_mesh = plsc.VectorSubcoreMesh(core_axis_name="c", subcore_axis_name="s")


@functools.partial(
    pl.kernel,
    out_type=jax.ShapeDtypeStruct((B, S, D), jnp.float32),
    mesh=_mesh,
    scratch_types=[
        pltpu.VMEM((NSEQ, HALVES, CH), jnp.int32),
        pltpu.VMEM((S, D), jnp.float32),
        [pltpu.VMEM((S, D), jnp.float32)] * NBUF,
        [pltpu.SemaphoreType.DMA] * NBUF,
        [pltpu.SemaphoreType.DMA] * NBUF,
    ],
    compiler_params=pltpu.CompilerParams(use_tc_tiling_on_sc=False),
)
def _embed_sc(token_hbm, word_hbm, pos_hbm, out_hbm,
              idx_v, pos_v, bufs, gsems, ssems):
    wid = lax.axis_index("s") * NC + lax.axis_index("c")
    b0 = wid * NSEQ

    pltpu.sync_copy(token_hbm.at[wid], idx_v)
    pltpu.sync_copy(pos_hbm, pos_v)

    def gather_start(seq, k):
        for h in range(HALVES):
            pltpu.async_copy(word_hbm.at[idx_v.at[seq, h]],
                             bufs[k].at[pl.ds(h * CH, CH)], gsems[k])

    def gather_wait(seq, k):
        for h in range(HALVES):
            pltpu.make_async_copy(word_hbm.at[idx_v.at[seq, h]],
                                  bufs[k].at[pl.ds(h * CH, CH)], gsems[k]).wait()

    def fma_rows(k):
        buf = bufs[k]
        def row(r, _):
            for c in range(VECS):
                sl = pl.ds(c * L, L)
                buf[r, sl] = buf[r, sl] * SCALE + pos_v[r, sl]
            return ()
        lax.fori_loop(0, S, row, (), unroll=4)

    def store_start(seq, k):
        pltpu.async_copy(bufs[k], out_hbm.at[b0 + seq], ssems[k])

    def store_wait(k):
        pltpu.make_async_copy(bufs[k], out_hbm.at[b0], ssems[k]).wait()

    for k in range(NBUF):
        gather_start(k, k)

    def body(i, _):
        j = NBUF * i
        for k in range(NBUF):
            gather_wait(j + k, k)
            fma_rows(k)
            store_start(j + k, k)
        for k in range(NBUF):
            @pl.when(j + NBUF + k < NSEQ)
            def _(k=k):
                store_wait(k)
                gather_start(j + NBUF + k, k)
        return ()

    lax.fori_loop(0, NSEQ // NBUF, body, ())

    for k in range(NBUF):
        store_wait(k)


def kernel(token, word_table, pos_table):
    tok = token.reshape(NW, NSEQ, HALVES, CH).astype(jnp.int32)
    return _embed_sc(tok, word_table, pos_table)
